# baseline (speedup 1.0000x reference)
import jax
import jax.numpy as jnp
from jax import lax
from jax.experimental import pallas as pl
from jax.experimental.pallas import tpu as pltpu
from jax.experimental.pallas import tpu_sc as plsc

NUM_MOLECULES = 16384
NUM_ATOMS = 200
NUM_SPECIES = 8

NC = 2
NS = 16
L = 16
NW = NC * NS
M_PER_W = NUM_MOLECULES // NW
WPM = NUM_ATOMS // 4
CH = 128
NCHUNK = M_PER_W // CH
NGROUP = CH // L


def _body(packed_hbm, energies_hbm, ptab_hbm, out_hbm,
          buf0, buf1, ptab_v, ebuf, obuf, sem0, sem1, esem):
    wid = lax.axis_index("s") * NC + lax.axis_index("c")
    base = wid * M_PER_W

    bufs = (buf0, buf1)
    sems = (sem0, sem1)
    pending = pltpu.async_copy(
        packed_hbm.at[pl.ds(base * WPM, CH * WPM)], buf0, sem0)
    ecopy = pltpu.async_copy(
        energies_hbm.at[pl.ds(base, M_PER_W)], ebuf, esem)
    pltpu.sync_copy(ptab_hbm, ptab_v)
    ecopy.wait()

    lanes = lax.iota(jnp.int32, L)

    for c in range(NCHUNK):
        nxt = None
        if c + 1 < NCHUNK:
            nxt = pltpu.async_copy(
                packed_hbm.at[pl.ds((base + (c + 1) * CH) * WPM, CH * WPM)],
                bufs[(c + 1) % 2], sems[(c + 1) % 2])
        pending.wait()
        buf = bufs[c % 2]

        def group_body(g, carry, buf=buf, c=c):
            wordbase = (g * L + lanes) * WPM
            acc0 = jnp.zeros((L,), jnp.float32)
            acc1 = jnp.zeros((L,), jnp.float32)
            for k in range(WPM):
                w = plsc.load_gather(buf, [wordbase + k])
                t0 = w & 7
                t1 = (w >> 8) & 7
                t2 = (w >> 16) & 7
                t3 = w >> 24
                i0 = (t0 << 7) | (t1 << 4) | lanes
                i1 = (t2 << 7) | (t3 << 4) | lanes
                acc0 = acc0 + plsc.load_gather(ptab_v, [i0])
                acc1 = acc1 + plsc.load_gather(ptab_v, [i1])
            off = c * CH + g * L
            obuf[pl.ds(off, L)] = (acc0 + acc1) + ebuf[pl.ds(off, L)]
            return carry

        lax.fori_loop(0, NGROUP, group_body, None)
        pending = nxt

    pltpu.sync_copy(obuf, out_hbm.at[pl.ds(base, M_PER_W)])


@jax.jit
def _shifter(packed, energies, ptab):
    mesh = plsc.VectorSubcoreMesh(
        core_axis_name="c", subcore_axis_name="s",
        num_cores=NC, num_subcores=NS)
    run = pl.kernel(
        _body,
        out_type=jax.ShapeDtypeStruct((NUM_MOLECULES,), jnp.float32),
        mesh=mesh,
        scratch_types=[
            pltpu.VMEM((CH * WPM,), jnp.int32),
            pltpu.VMEM((CH * WPM,), jnp.int32),
            pltpu.VMEM((NUM_SPECIES * NUM_SPECIES * L,), jnp.float32),
            pltpu.VMEM((M_PER_W,), jnp.float32),
            pltpu.VMEM((M_PER_W,), jnp.float32),
            pltpu.SemaphoreType.DMA,
            pltpu.SemaphoreType.DMA,
            pltpu.SemaphoreType.DMA,
        ],
        compiler_params=pltpu.CompilerParams(
            use_tc_tiling_on_sc=False, needs_layout_passes=False),
    )
    return run(packed, energies, ptab)


def kernel(species, energies, self_energies):
    t = self_energies.astype(jnp.float32)
    ptab = jnp.broadcast_to(
        (t[:, None] + t[None, :])[:, :, None],
        (NUM_SPECIES, NUM_SPECIES, L)).reshape(-1)
    packed = lax.bitcast_convert_type(
        species.astype(jnp.int8).reshape(NUM_MOLECULES * WPM, 4),
        jnp.int32)
    shifted = _shifter(packed, energies, ptab)
    return species, shifted


# speedup vs baseline: 4.2393x; 4.2393x over previous
import jax
import jax.numpy as jnp
from jax import lax
from jax.experimental import pallas as pl
from jax.experimental.pallas import tpu as pltpu
from jax.experimental.pallas import tpu_sc as plsc

NUM_MOLECULES = 16384
NUM_ATOMS = 200
NUM_SPECIES = 8
HALF = NUM_ATOMS // 2

NC = 2
NS = 16
L = 16
NW = NC * NS
M_PER_W = NUM_MOLECULES // NW
CH = 128
NCHUNK = M_PER_W // CH
NGROUP = CH // L


def _body(species_hbm, energies_hbm, ptab_hbm, out_hbm,
          buf0, buf1, ptab_v, ebuf, obuf, sem0, sem1, esem):
    wid = lax.axis_index("s") * NC + lax.axis_index("c")
    base = wid * M_PER_W

    bufs = (buf0, buf1)
    sems = (sem0, sem1)
    pending = pltpu.async_copy(species_hbm.at[pl.ds(base, CH)], buf0, sem0)
    ecopy = pltpu.async_copy(
        energies_hbm.at[pl.ds(base, M_PER_W)], ebuf, esem)
    pltpu.sync_copy(ptab_hbm, ptab_v)
    ecopy.wait()

    lanes = lax.iota(jnp.int32, L)

    for c in range(NCHUNK):
        nxt = None
        if c + 1 < NCHUNK:
            nxt = pltpu.async_copy(
                species_hbm.at[pl.ds(base + (c + 1) * CH, CH)],
                bufs[(c + 1) % 2], sems[(c + 1) % 2])
        pending.wait()
        buf = bufs[c % 2]

        def group_body(g, carry, buf=buf, c=c):
            rows = g * L + lanes
            acc0 = jnp.zeros((L,), jnp.float32)
            acc1 = jnp.zeros((L,), jnp.float32)
            cols = lanes
            for a in range(HALF):
                s_lo = plsc.load_gather(buf, [rows, cols])
                s_hi = plsc.load_gather(buf, [rows, cols + HALF])
                idx = (s_lo << 7) | (s_hi << 4) | lanes
                v = plsc.load_gather(ptab_v, [idx])
                if a % 2 == 0:
                    acc0 = acc0 + v
                else:
                    acc1 = acc1 + v
                ncols = cols + 1
                cols = jnp.where(ncols >= HALF, ncols - HALF, ncols)
            off = c * CH + g * L
            obuf[pl.ds(off, L)] = (acc0 + acc1) + ebuf[pl.ds(off, L)]
            return carry

        lax.fori_loop(0, NGROUP, group_body, None)
        pending = nxt

    pltpu.sync_copy(obuf, out_hbm.at[pl.ds(base, M_PER_W)])


@jax.jit
def _shifter(species, energies, ptab):
    mesh = plsc.VectorSubcoreMesh(
        core_axis_name="c", subcore_axis_name="s",
        num_cores=NC, num_subcores=NS)
    run = pl.kernel(
        _body,
        out_type=jax.ShapeDtypeStruct((NUM_MOLECULES,), jnp.float32),
        mesh=mesh,
        scratch_types=[
            pltpu.VMEM((CH, NUM_ATOMS), jnp.int32),
            pltpu.VMEM((CH, NUM_ATOMS), jnp.int32),
            pltpu.VMEM((NUM_SPECIES * NUM_SPECIES * L,), jnp.float32),
            pltpu.VMEM((M_PER_W,), jnp.float32),
            pltpu.VMEM((M_PER_W,), jnp.float32),
            pltpu.SemaphoreType.DMA,
            pltpu.SemaphoreType.DMA,
            pltpu.SemaphoreType.DMA,
        ],
        compiler_params=pltpu.CompilerParams(
            use_tc_tiling_on_sc=False, needs_layout_passes=False),
    )
    return run(species, energies, ptab)


def kernel(species, energies, self_energies):
    t = self_energies.astype(jnp.float32)
    ptab = jnp.broadcast_to(
        (t[:, None] + t[None, :])[:, :, None],
        (NUM_SPECIES, NUM_SPECIES, L)).reshape(-1)
    shifted = _shifter(species, energies, ptab)
    return species, shifted


# speedup vs baseline: 10.4089x; 2.4553x over previous
import jax
import jax.numpy as jnp
from jax import lax
from jax.experimental import pallas as pl
from jax.experimental.pallas import tpu as pltpu
from jax.experimental.pallas import tpu_sc as plsc

NUM_MOLECULES = 16384
NUM_ATOMS = 200
NUM_SPECIES = 8
HALF = NUM_ATOMS // 2

NC = 2
NS = 16
L = 16
NW = NC * NS
M_PER_W = NUM_MOLECULES // NW
CH = 128
NCHUNK = M_PER_W // CH
NGROUP = CH // L


def _body(speciesT_hbm, energies_hbm, ptab_hbm, out_hbm,
          buf0, buf1, ptab_v, ebuf, obuf, sem0, sem1, esem):
    wid = lax.axis_index("s") * NC + lax.axis_index("c")
    base = wid * M_PER_W

    bufs = (buf0, buf1)
    sems = (sem0, sem1)
    pending = pltpu.async_copy(
        speciesT_hbm.at[:, pl.ds(base, CH)], buf0, sem0)
    ecopy = pltpu.async_copy(
        energies_hbm.at[pl.ds(base, M_PER_W)], ebuf, esem)
    pltpu.sync_copy(ptab_hbm, ptab_v)
    ecopy.wait()

    lanes = lax.iota(jnp.int32, L)

    for c in range(NCHUNK):
        nxt = None
        if c + 1 < NCHUNK:
            nxt = pltpu.async_copy(
                speciesT_hbm.at[:, pl.ds(base + (c + 1) * CH, CH)],
                bufs[(c + 1) % 2], sems[(c + 1) % 2])
        pending.wait()
        buf = bufs[c % 2]

        def group_body(g, carry, buf=buf, c=c):
            off = g * L
            acc0 = jnp.zeros((L,), jnp.float32)
            acc1 = jnp.zeros((L,), jnp.float32)
            for a in range(HALF):
                s_lo = buf[a, pl.ds(off, L)]
                s_hi = buf[a + HALF, pl.ds(off, L)]
                idx = (s_lo << 7) | (s_hi << 4) | lanes
                v = plsc.load_gather(ptab_v, [idx])
                if a % 2 == 0:
                    acc0 = acc0 + v
                else:
                    acc1 = acc1 + v
            oat = c * CH + off
            obuf[pl.ds(oat, L)] = (acc0 + acc1) + ebuf[pl.ds(oat, L)]
            return carry

        lax.fori_loop(0, NGROUP, group_body, None)
        pending = nxt

    pltpu.sync_copy(obuf, out_hbm.at[pl.ds(base, M_PER_W)])


@jax.jit
def _shifter(speciesT, energies, ptab):
    mesh = plsc.VectorSubcoreMesh(
        core_axis_name="c", subcore_axis_name="s",
        num_cores=NC, num_subcores=NS)
    run = pl.kernel(
        _body,
        out_type=jax.ShapeDtypeStruct((NUM_MOLECULES,), jnp.float32),
        mesh=mesh,
        scratch_types=[
            pltpu.VMEM((NUM_ATOMS, CH), jnp.int32),
            pltpu.VMEM((NUM_ATOMS, CH), jnp.int32),
            pltpu.VMEM((NUM_SPECIES * NUM_SPECIES * L,), jnp.float32),
            pltpu.VMEM((M_PER_W,), jnp.float32),
            pltpu.VMEM((M_PER_W,), jnp.float32),
            pltpu.SemaphoreType.DMA,
            pltpu.SemaphoreType.DMA,
            pltpu.SemaphoreType.DMA,
        ],
        compiler_params=pltpu.CompilerParams(
            use_tc_tiling_on_sc=True, needs_layout_passes=False),
    )
    return run(speciesT, energies, ptab)


def kernel(species, energies, self_energies):
    t = self_energies.astype(jnp.float32)
    ptab = jnp.broadcast_to(
        (t[:, None] + t[None, :])[:, :, None],
        (NUM_SPECIES, NUM_SPECIES, L)).reshape(-1)
    shifted = _shifter(species.T, energies, ptab)
    return species, shifted
